# baseline (speedup 1.0000x reference)
import jax
import jax.numpy as jnp
from jax.experimental import pallas as pl
from jax.experimental.pallas import tpu as pltpu

_BLOCK = 2048
_D = 256


def _quant4(v, c_ref):
    c0 = c_ref[0]
    c1 = c_ref[1]
    c2 = c_ref[2]
    c3 = c_ref[3]
    m0 = 0.5 * (c0 + c1)
    m1 = 0.5 * (c1 + c2)
    m2 = 0.5 * (c2 + c3)
    b0 = v > m0
    b1 = v > m1
    b2 = v > m2
    idx = b0.astype(jnp.int32) + b1.astype(jnp.int32) + b2.astype(jnp.int32)
    val = jnp.where(b1, jnp.where(b2, c3, c2), jnp.where(b0, c1, c0))
    return idx, val


_CHUNK = 1024


def _rvq_kernel(x_ref, pi_ref, c1_ref, c2_ref,
                xhat_ref, i1_ref, i2_ref, norm_ref):
    pi = pi_ref[...]
    ones = jnp.ones((_D, 1), dtype=jnp.float32)
    for k in range(_BLOCK // _CHUNK):
        rows = pl.ds(k * _CHUNK, _CHUNK)
        x = x_ref[rows, :]
        norm = jnp.sqrt(jnp.sum(x * x, axis=1, keepdims=True))
        xn = x / (norm + 1e-8)
        xr = jax.lax.dot_general(xn, pi, (((1,), (1,)), ((), ())),
                                 preferred_element_type=jnp.float32)
        i1, xh1 = _quant4(xr, c1_ref)
        resid = xr - xh1
        i2, rh = _quant4(resid, c2_ref)
        q = xh1 + rh
        xc = jax.lax.dot_general(q, pi, (((1,), (0,)), ((), ())),
                                 preferred_element_type=jnp.float32)
        xhat_ref[rows, :] = xc * norm
        i1_ref[rows, :] = i1
        i2_ref[rows, :] = i2
        norm_ref[rows] = norm[:, 0]


def kernel(x, Pi, centroids1, centroids2):
    n, d = x.shape
    out = pl.pallas_call(
        _rvq_kernel,
        grid=(n // _BLOCK,),
        in_specs=[
            pl.BlockSpec((_BLOCK, d), lambda i: (i, 0)),
            pl.BlockSpec((d, d), lambda i: (0, 0)),
            pl.BlockSpec(memory_space=pltpu.SMEM),
            pl.BlockSpec(memory_space=pltpu.SMEM),
        ],
        out_specs=[
            pl.BlockSpec((_BLOCK, d), lambda i: (i, 0)),
            pl.BlockSpec((_BLOCK, d), lambda i: (i, 0)),
            pl.BlockSpec((_BLOCK, d), lambda i: (i, 0)),
            pl.BlockSpec((_BLOCK,), lambda i: (i,)),
        ],
        out_shape=[
            jax.ShapeDtypeStruct((n, d), jnp.float32),
            jax.ShapeDtypeStruct((n, d), jnp.int32),
            jax.ShapeDtypeStruct((n, d), jnp.int32),
            jax.ShapeDtypeStruct((n,), jnp.float32),
        ],
        compiler_params=pltpu.CompilerParams(
            dimension_semantics=("parallel",)),
    )(x, Pi, centroids1, centroids2)
    return (out[0], out[1], out[2], out[3])


# speedup vs baseline: 1.2683x; 1.2683x over previous
import jax
import jax.numpy as jnp
from jax.experimental import pallas as pl
from jax.experimental.pallas import tpu as pltpu

_BLOCK = 4096
_D = 256


def _quant4(v, c_ref):
    c0 = c_ref[0]
    c1 = c_ref[1]
    c2 = c_ref[2]
    c3 = c_ref[3]
    m0 = 0.5 * (c0 + c1)
    m1 = 0.5 * (c1 + c2)
    m2 = 0.5 * (c2 + c3)
    b0 = v > m0
    b1 = v > m1
    b2 = v > m2
    one = jnp.int32(1)
    idx = jnp.where(b1, jnp.where(b2, one + 2, one + 1),
                    jnp.where(b0, one, one - 1))
    val = jnp.where(b1, jnp.where(b2, c3, c2), jnp.where(b0, c1, c0))
    return idx, val


_CHUNK = 1024


def _rvq_kernel(x_ref, pi_ref, c1_ref, c2_ref,
                xhat_ref, i1_ref, i2_ref, norm_ref):
    pi = pi_ref[...]
    ident = jnp.eye(128, dtype=jnp.float32)
    for k in range(_BLOCK // _CHUNK):
        rows = pl.ds(k * _CHUNK, _CHUNK)
        x = x_ref[rows, :]
        norm = jnp.sqrt(jnp.sum(x * x, axis=1, keepdims=True))
        xn = x / (norm + 1e-8)
        xr = jax.lax.dot_general(xn, pi, (((1,), (1,)), ((), ())),
                                 preferred_element_type=jnp.float32)
        i1, xh1 = _quant4(xr, c1_ref)
        resid = xr - xh1
        i2, rh = _quant4(resid, c2_ref)
        q = xh1 + rh
        xc = jax.lax.dot_general(q, pi, (((1,), (0,)), ((), ())),
                                 preferred_element_type=jnp.float32)
        xhat_ref[rows, :] = xc * norm
        i1_ref[rows, :] = i1
        i2_ref[rows, :] = i2
        flat = jnp.concatenate(
            [jax.lax.dot_general(norm[g * 128:(g + 1) * 128, :], ident,
                                 (((0,), (0,)), ((), ())),
                                 preferred_element_type=jnp.float32)
             for g in range(_CHUNK // 128)], axis=0)
        norm_ref[rows] = flat.reshape(_CHUNK)


def kernel(x, Pi, centroids1, centroids2):
    n, d = x.shape
    out = pl.pallas_call(
        _rvq_kernel,
        grid=(n // _BLOCK,),
        in_specs=[
            pl.BlockSpec((_BLOCK, d), lambda i: (i, 0)),
            pl.BlockSpec((d, d), lambda i: (0, 0)),
            pl.BlockSpec(memory_space=pltpu.SMEM),
            pl.BlockSpec(memory_space=pltpu.SMEM),
        ],
        out_specs=[
            pl.BlockSpec((_BLOCK, d), lambda i: (i, 0)),
            pl.BlockSpec((_BLOCK, d), lambda i: (i, 0)),
            pl.BlockSpec((_BLOCK, d), lambda i: (i, 0)),
            pl.BlockSpec((_BLOCK,), lambda i: (i,)),
        ],
        out_shape=[
            jax.ShapeDtypeStruct((n, d), jnp.float32),
            jax.ShapeDtypeStruct((n, d), jnp.int32),
            jax.ShapeDtypeStruct((n, d), jnp.int32),
            jax.ShapeDtypeStruct((n,), jnp.float32),
        ],
        compiler_params=pltpu.CompilerParams(
            dimension_semantics=("parallel",)),
    )(x, Pi, centroids1, centroids2)
    return (out[0], out[1], out[2], out[3])
